# baseline (speedup 1.0000x reference)
import functools

import jax
import jax.numpy as jnp
from jax import lax
from jax.experimental import pallas as pl
from jax.experimental.pallas import tpu as pltpu
from jax.experimental.pallas import tpu_sc as plsc

N = 8192
NF = 256
Q_BLK = 1024
R_CHUNK = 2048
BIG_I32 = 2**30


def _knn_body(t2t_ref, t1_ref, out_ref):
    q = t2t_ref[...]
    qn = (q[:, 0:1] * q[:, 0:1] + q[:, 1:2] * q[:, 1:2]) + q[:, 2:3] * q[:, 2:3]

    def body(c, carry):
        best, besti = carry
        t1c = t1_ref[:, pl.ds(c * R_CHUNK, R_CHUNK)]
        rn = (t1c[0:1] * t1c[0:1] + t1c[1:2] * t1c[1:2]) + t1c[2:3] * t1c[2:3]
        cr = lax.dot_general(q.astype(jnp.bfloat16), t1c.astype(jnp.bfloat16),
                             (((1,), (0,)), ((), ())),
                             preferred_element_type=jnp.float32)
        d = (qn + rn) - 2.0 * cr
        m = jnp.min(d, axis=1, keepdims=True)
        io = lax.broadcasted_iota(jnp.int32, (Q_BLK, R_CHUNK), 1) + c * R_CHUNK
        i = jnp.min(jnp.where(d == m, io, BIG_I32), axis=1, keepdims=True)
        upd = m < best
        best = jnp.where(upd, m, best)
        rounded = best.astype(jnp.bfloat16).astype(jnp.float32)
        best = jnp.where((c % 2) == 1, rounded, best)
        return best, jnp.where(upd, i, besti)

    init = (jnp.full((Q_BLK, 1), jnp.inf, jnp.float32),
            jnp.zeros((Q_BLK, 1), jnp.int32))
    _, besti = lax.fori_loop(0, N // R_CHUNK, body, init)
    out_ref[...] = besti


_knn = pl.pallas_call(
    _knn_body,
    grid=(N // Q_BLK,),
    in_specs=[
        pl.BlockSpec((Q_BLK, 3), lambda i: (i, 0)),
        pl.BlockSpec((3, N), lambda i: (0, 0)),
    ],
    out_specs=pl.BlockSpec((Q_BLK, 1), lambda i: (i, 0)),
    out_shape=jax.ShapeDtypeStruct((N, 1), jnp.int32),
)


_NC, _NS = 2, 16
_NW = _NC * _NS
_ROWS_PER_W = NF // _NW

_sc_mesh = plsc.VectorSubcoreMesh(core_axis_name="c", subcore_axis_name="s")


@functools.partial(
    pl.kernel,
    out_type=jax.ShapeDtypeStruct((2 * NF, N), jnp.float32),
    mesh=_sc_mesh,
    scratch_types=[
        pltpu.VMEM((N,), jnp.int32),
        pltpu.VMEM((N,), jnp.float32),
        pltpu.VMEM((N,), jnp.float32),
    ],
    compiler_params=pltpu.CompilerParams(needs_layout_passes=False),
)
def _gather_sc(emb1_hbm, emb2_hbm, idx_hbm, out_hbm, idx_v, row_v, orow_v):
    wid = lax.axis_index("s") * _NC + lax.axis_index("c")
    pltpu.sync_copy(idx_hbm, idx_v)

    def row_body(j, _):
        r = wid * _ROWS_PER_W + j
        pltpu.sync_copy(emb1_hbm.at[r], row_v)

        def qloop(qi, _):
            iv = idx_v[pl.ds(qi * 16, 16)]
            orow_v[pl.ds(qi * 16, 16)] = plsc.load_gather(row_v, [iv])
            return 0

        lax.fori_loop(0, N // 16, qloop, 0, unroll=8)
        pltpu.sync_copy(orow_v, out_hbm.at[r])
        pltpu.sync_copy(emb2_hbm.at[r], row_v)
        pltpu.sync_copy(row_v, out_hbm.at[NF + r])
        return 0

    lax.fori_loop(0, _ROWS_PER_W, row_body, 0)


def kernel(emb1, emb2, t1, t2):
    t1_2 = t1.reshape(3, N)
    t2_2 = t2.reshape(3, N)
    idx = _knn(t2_2.T, t1_2).reshape(N)
    out = _gather_sc(emb1.reshape(NF, N), emb2.reshape(NF, N), idx)
    return out.reshape(1, 2 * NF, N)
